# baseline (speedup 1.0000x reference)
import functools

import jax
import jax.numpy as jnp
from jax import lax
from jax.experimental import pallas as pl
from jax.experimental.pallas import tpu as pltpu
from jax.experimental.pallas import tpu_sc as plsc

_NC = 2
_NS = 16
_NW = _NC * _NS


def _make_deg(E, NP):
    EC = E // _NW
    K = 80
    STEPS = EC // K
    NPS = NP // _NS
    mesh = plsc.VectorSubcoreMesh(core_axis_name="c", subcore_axis_name="s")

    @functools.partial(
        pl.kernel,
        out_type=jax.ShapeDtypeStruct((_NC, 1, NP), jnp.float32),
        mesh=mesh,
        scratch_types=[
            pltpu.VMEM((K,), jnp.int32),
            pltpu.VMEM((K,), jnp.float32),
            pltpu.VMEM((NPS,), jnp.float32),
            pltpu.VMEM_SHARED((NP,), jnp.float32),
        ],
    )
    def k(dst_hbm, deg_out, idv, onesv, dbuf, acc):
        c = lax.axis_index("c")
        s = lax.axis_index("s")
        wid = s * _NC + c

        for j in range(K // 16):
            onesv[pl.ds(j * 16, 16)] = jnp.ones((16,), jnp.float32)

        def zero(i, carry):
            dbuf[pl.ds(i * 16, 16)] = jnp.zeros((16,), jnp.float32)
            return carry
        lax.fori_loop(0, NPS // 16, zero, 0)
        pltpu.sync_copy(dbuf, acc.at[pl.ds(s * NPS, NPS)])
        plsc.subcore_barrier()

        def body(i, carry):
            off = wid * EC + i * K
            pltpu.sync_copy(dst_hbm.at[pl.ds(off, K)], idv)
            pltpu.sync_copy(onesv, acc.at[idv], add=True)
            return carry
        lax.fori_loop(0, STEPS, body, 0)

        plsc.subcore_barrier()
        pltpu.sync_copy(acc.at[pl.ds(s * NPS, NPS)], dbuf)
        pltpu.sync_copy(dbuf, deg_out.at[c, 0, pl.ds(s * NPS, NPS)])

    return k


def _make_spmm(E, NV2, F):
    K = 80
    STEPS = 128
    CH = 16
    NCH = STEPS // CH
    RPT = NV2 // _NS
    DR = 80
    mesh = plsc.VectorSubcoreMesh(core_axis_name="c", subcore_axis_name="s")

    @functools.partial(
        pl.kernel,
        out_type=jax.ShapeDtypeStruct((_NC, NV2, F), jnp.float32),
        mesh=mesh,
        scratch_types=[
            pltpu.VMEM((CH, K), jnp.int32),
            pltpu.VMEM((CH, K), jnp.int32),
            pltpu.VMEM((K, F), jnp.float32),
            pltpu.VMEM((K, F), jnp.float32),
            pltpu.VMEM_SHARED((NV2, F), jnp.float32),
            pltpu.SemaphoreType.DMA,
            pltpu.SemaphoreType.DMA,
            pltpu.SemaphoreType.DMA,
            pltpu.SemaphoreType.DMA,
        ],
    )
    def k(m_hbm, src_hbm, dst_hbm, out_hbm, srcv, dstv, rows0, rows1,
          acc, sem0, sem1, ssem0, ssem1):
        c = lax.axis_index("c")
        s = lax.axis_index("s")
        wid = s * _NC + c

        def zero(i, carry):
            r = i // (F // 16)
            j = i - r * (F // 16)
            rows0[r, pl.ds(j * 16, 16)] = jnp.zeros((16,), jnp.float32)
            return carry
        lax.fori_loop(0, DR * (F // 16), zero, 0)

        r0 = s * RPT
        for b in range(RPT // DR):
            pltpu.sync_copy(rows0, acc.at[pl.ds(r0 + b * DR, DR)])
        plsc.subcore_barrier()

        def wait_g(i, buf, sem):
            pltpu.make_async_copy(m_hbm.at[srcv.at[i]], buf, sem).wait()

        def wait_s(i, buf, sem):
            pltpu.make_async_copy(buf, acc.at[dstv.at[i]], sem).wait()

        for ch in range(NCH):
            pltpu.sync_copy(src_hbm.at[wid, pl.ds(ch * CH, CH)], srcv)
            pltpu.sync_copy(dst_hbm.at[wid, pl.ds(ch * CH, CH)], dstv)
            pltpu.async_copy(m_hbm.at[srcv.at[0]], rows0, sem0)
            pltpu.async_copy(m_hbm.at[srcv.at[1]], rows1, sem1)

            def step2(g, carry):
                i = 2 * g
                wait_g(i, rows0, sem0)
                pltpu.async_copy(rows0, acc.at[dstv.at[i]], ssem0,
                                 add=True)
                wait_g(i + 1, rows1, sem1)
                pltpu.async_copy(rows1, acc.at[dstv.at[i + 1]], ssem1,
                                 add=True)
                wait_s(i, rows0, ssem0)
                pltpu.async_copy(m_hbm.at[srcv.at[i + 2]], rows0, sem0)
                wait_s(i + 1, rows1, ssem1)
                pltpu.async_copy(m_hbm.at[srcv.at[i + 3]], rows1, sem1)
                return carry
            lax.fori_loop(0, CH // 2 - 1, step2, 0)

            wait_g(CH - 2, rows0, sem0)
            pltpu.async_copy(rows0, acc.at[dstv.at[CH - 2]], ssem0,
                             add=True)
            wait_g(CH - 1, rows1, sem1)
            pltpu.async_copy(rows1, acc.at[dstv.at[CH - 1]], ssem1,
                             add=True)
            wait_s(CH - 2, rows0, ssem0)
            wait_s(CH - 1, rows1, ssem1)

        plsc.subcore_barrier()
        for b in range(RPT // DR):
            rr = r0 + b * DR
            pltpu.sync_copy(acc.at[pl.ds(rr, DR)], rows0)
            pltpu.sync_copy(rows0, out_hbm.at[c, pl.ds(rr, DR)])

    return k


def _rsqrt_body(d0_ref, d1_ref, o_ref):
    o_ref[...] = 1.0 / jnp.sqrt(d0_ref[...] + d1_ref[...] + 1.0)


def _dinv_tc(deg0, deg1):
    return pl.pallas_call(
        _rsqrt_body,
        out_shape=jax.ShapeDtypeStruct(deg0.shape, jnp.float32),
    )(deg0, deg1)


def _mm1_body(x_ref, dv_ref, w_ref, we_ref, be_ref, m_ref, re_ref):
    xb = x_ref[...]
    m_ref[...] = jnp.dot(xb, w_ref[...],
                         preferred_element_type=jnp.float32) * dv_ref[...]
    re_ref[...] = jnp.maximum(
        jnp.dot(xb, we_ref[...], preferred_element_type=jnp.float32)
        + be_ref[...], 0.0)


def _mm1_tc(x, dinv_b, W, We, be_r):
    N, F = x.shape
    R = 1000
    grid = (N // R,)
    blk = pl.BlockSpec((R, F), lambda i: (i, 0))
    wblk = pl.BlockSpec((F, F), lambda i: (0, 0))
    return pl.pallas_call(
        _mm1_body,
        grid=grid,
        in_specs=[blk, blk, wblk, wblk, pl.BlockSpec((1, F), lambda i: (0, 0))],
        out_specs=[blk, blk],
        out_shape=[jax.ShapeDtypeStruct((N, F), jnp.float32)] * 2,
    )(x, dinv_b, W, We, be_r)


def _comb_body(a0_ref, a1_ref, m_ref, re_ref, dv_ref, b_ref, w2_ref,
               h_ref, m2_ref):
    dv = dv_ref[...]
    s = (a0_ref[0] + a1_ref[0] + m_ref[...]) * dv + b_ref[...]
    h = jnp.maximum(s, 0.0) + re_ref[...]
    h_ref[...] = h
    m2_ref[...] = jnp.dot(h, w2_ref[...],
                          preferred_element_type=jnp.float32) * dv


def _comb_tc(acc, m, re, dinv_b, b_r, W2):
    N, F = m.shape
    R = 1000
    blk = pl.BlockSpec((R, F), lambda i: (i, 0))
    ablk0 = pl.BlockSpec((1, R, F), lambda i: (0, i, 0))
    ablk1 = pl.BlockSpec((1, R, F), lambda i: (1, i, 0))
    return pl.pallas_call(
        _comb_body,
        grid=(N // R,),
        in_specs=[ablk0, ablk1, blk, blk, blk,
                  pl.BlockSpec((1, F), lambda i: (0, 0)),
                  pl.BlockSpec((F, F), lambda i: (0, 0))],
        out_specs=[blk, blk],
        out_shape=[jax.ShapeDtypeStruct((N, F), jnp.float32)] * 2,
    )(acc, acc, m, re, dinv_b, b_r, W2)


def _final_body(a0_ref, a1_ref, m2_ref, h_ref, b2_ref,
                a0s_ref, a1s_ref, m2s_ref, hs_ref, b2s_ref,
                dv_ref, batch_ref, wf_ref, bf_ref, o_ref,
                seg_ref, cnt_ref):
    i = pl.program_id(0)
    n = pl.num_programs(0)
    G = seg_ref.shape[0]
    R = h_ref.shape[0]

    @pl.when(i == 0)
    def _():
        seg_ref[...] = jnp.zeros_like(seg_ref)
        cnt_ref[...] = jnp.zeros_like(cnt_ref)

    dv = dv_ref[...]
    h2 = jnp.maximum((a0_ref[0] + a1_ref[0] + m2_ref[...]) * dv
                     + b2_ref[...], 0.0) + h_ref[...]
    h2s = jnp.maximum((a0s_ref[0] + a1s_ref[0] + m2s_ref[...]) * dv
                      + b2s_ref[...], 0.0) + hs_ref[...]
    ht = h2 + h2s

    b = batch_ref[0]
    gid = lax.broadcasted_iota(jnp.int32, (G, R), 0)
    oh = (gid == jnp.broadcast_to(b, (G, R))).astype(jnp.float32)
    seg_ref[...] += jnp.dot(oh, ht, preferred_element_type=jnp.float32,
                         precision=lax.Precision.HIGHEST)
    cnt_ref[...] += jnp.broadcast_to(
        jnp.sum(oh, axis=1, keepdims=True), cnt_ref.shape)

    @pl.when(i == n - 1)
    def _():
        pooled = seg_ref[...] / jnp.maximum(cnt_ref[...], 1.0)
        pooled = pooled.astype(jnp.bfloat16).astype(jnp.float32)
        wf = wf_ref[...].astype(jnp.bfloat16).astype(jnp.float32)
        pred = jnp.sum(pooled * wf, axis=1, keepdims=True)
        o_ref[...] = pred + bf_ref[0, 0]


def _final_tc(acc2, m2, h, b2_r, acc2s, m2s, hs, b2s_r,
              dinv_b, batch3, wf_r, bf_b, G):
    N, F = h.shape
    R = 1000
    blk = pl.BlockSpec((R, F), lambda i: (i, 0))
    cblk = pl.BlockSpec((1, F), lambda i: (0, 0))
    ablk0 = pl.BlockSpec((1, R, F), lambda i: (0, i, 0))
    ablk1 = pl.BlockSpec((1, R, F), lambda i: (1, i, 0))
    return pl.pallas_call(
        _final_body,
        grid=(N // R,),
        in_specs=[ablk0, ablk1, blk, blk, cblk,
                  ablk0, ablk1, blk, blk, cblk,
                  blk,
                  pl.BlockSpec((1, 1, R), lambda i: (i, 0, 0)),
                  cblk, cblk],
        out_specs=pl.BlockSpec((G, 1), lambda i: (0, 0)),
        out_shape=jax.ShapeDtypeStruct((G, 1), jnp.float32),
        scratch_shapes=[pltpu.VMEM((G, F), jnp.float32),
                        pltpu.VMEM((G, F), jnp.float32)],
    )(acc2, acc2, m2, h, b2_r, acc2s, acc2s, m2s, hs, b2s_r, dinv_b, batch3,
      wf_r, bf_b)


def kernel(x, x_SC, edge_index, edge_weight, batch,
           W1, b1, W2, b2, We, be,
           W1s, b1s, W2s, b2s, Wes, bes,
           Wf, bf):
    N, F = x.shape
    E = edge_index.shape[1]
    G = 64
    NP = 10240

    src = edge_index[0].astype(jnp.int32)
    dst = edge_index[1].astype(jnp.int32)
    E2 = _NW * 128 * 80
    src3 = jnp.concatenate(
        [src, jnp.zeros((E2 - E,), jnp.int32)]).reshape(_NW, 128, 80)
    dst3 = jnp.concatenate(
        [dst, jnp.full((E2 - E,), NP - 1, jnp.int32)]).reshape(_NW, 128, 80)

    deg3 = _make_deg(E, NP)(dst)
    dinv80 = _dinv_tc(deg3[0, 0].reshape(NP // F, F),
                      deg3[1, 0].reshape(NP // F, F))
    dinv_b = jnp.broadcast_to(
        dinv80.reshape(NP)[:N][:, None], (N, F))

    spmm = _make_spmm(E, NP, F)

    m1, re1 = _mm1_tc(x, dinv_b, W1, We, be.reshape(1, F))
    m1s, re1s = _mm1_tc(x_SC, dinv_b, W1s, Wes, bes.reshape(1, F))

    acc1 = spmm(m1, src3, dst3)
    acc1s = spmm(m1s, src3, dst3)

    h1, m2 = _comb_tc(acc1, m1, re1, dinv_b, b1.reshape(1, F), W2)
    h1s, m2s = _comb_tc(acc1s, m1s, re1s, dinv_b, b1s.reshape(1, F), W2s)

    acc2 = spmm(m2, src3, dst3)
    acc2s = spmm(m2s, src3, dst3)

    batch3 = batch.astype(jnp.int32).reshape(N // 1000, 1, 1000)
    out = _final_tc(acc2, m2, h1, b2.reshape(1, F),
                    acc2s, m2s, h1s, b2s.reshape(1, F),
                    dinv_b, batch3, Wf.reshape(1, F),
                    jnp.broadcast_to(bf.reshape(1, 1), (1, F)), G)
    return out


# speedup vs baseline: 1.7057x; 1.7057x over previous
import functools

import jax
import jax.numpy as jnp
from jax import lax
from jax.experimental import pallas as pl
from jax.experimental.pallas import tpu as pltpu
from jax.experimental.pallas import tpu_sc as plsc

_NC = 2
_NS = 16
_NW = _NC * _NS


def _make_deg(E, NP):
    EC = E // _NW
    K = 80
    STEPS = EC // K
    NPS = NP // _NS
    mesh = plsc.VectorSubcoreMesh(core_axis_name="c", subcore_axis_name="s")

    @functools.partial(
        pl.kernel,
        out_type=jax.ShapeDtypeStruct((_NC, 1, NP), jnp.float32),
        mesh=mesh,
        scratch_types=[
            pltpu.VMEM((K,), jnp.int32),
            pltpu.VMEM((K,), jnp.float32),
            pltpu.VMEM((NPS,), jnp.float32),
            pltpu.VMEM_SHARED((NP,), jnp.float32),
        ],
    )
    def k(dst_hbm, deg_out, idv, onesv, dbuf, acc):
        c = lax.axis_index("c")
        s = lax.axis_index("s")
        wid = s * _NC + c

        for j in range(K // 16):
            onesv[pl.ds(j * 16, 16)] = jnp.ones((16,), jnp.float32)

        def zero(i, carry):
            dbuf[pl.ds(i * 16, 16)] = jnp.zeros((16,), jnp.float32)
            return carry
        lax.fori_loop(0, NPS // 16, zero, 0)
        pltpu.sync_copy(dbuf, acc.at[pl.ds(s * NPS, NPS)])
        plsc.subcore_barrier()

        def body(i, carry):
            off = wid * EC + i * K
            pltpu.sync_copy(dst_hbm.at[pl.ds(off, K)], idv)
            pltpu.sync_copy(onesv, acc.at[idv], add=True)
            return carry
        lax.fori_loop(0, STEPS, body, 0)

        plsc.subcore_barrier()
        pltpu.sync_copy(acc.at[pl.ds(s * NPS, NPS)], dbuf)
        pltpu.sync_copy(dbuf, deg_out.at[c, 0, pl.ds(s * NPS, NPS)])

    return k


def _make_spmm(E, NV2, F):
    EC = E // _NW
    K = 80
    STEPS = EC // K
    RPT = NV2 // _NS
    DR = 128
    mesh = plsc.VectorSubcoreMesh(core_axis_name="c", subcore_axis_name="s")

    @functools.partial(
        pl.kernel,
        out_type=jax.ShapeDtypeStruct((_NC, NV2, F), jnp.float32),
        mesh=mesh,
        scratch_types=[
            pltpu.VMEM((K,), jnp.int32),
            pltpu.VMEM((K,), jnp.int32),
            pltpu.VMEM((K, F), jnp.float32),
            pltpu.VMEM((DR, F), jnp.float32),
            pltpu.VMEM_SHARED((NV2, F), jnp.float32),
            pltpu.SemaphoreType.DMA,
        ],
    )
    def k(m_hbm, src_hbm, dst_hbm, out_hbm, isv, idv, rows, dbuf, acc,
          sem):
        c = lax.axis_index("c")
        s = lax.axis_index("s")
        wid = s * _NC + c

        def zero(i, carry):
            r = i // (F // 16)
            j = i - r * (F // 16)
            dbuf[r, pl.ds(j * 16, 16)] = jnp.zeros((16,), jnp.float32)
            return carry
        lax.fori_loop(0, DR * (F // 16), zero, 0)

        r0 = s * RPT
        for b in range(RPT // DR):
            pltpu.sync_copy(dbuf, acc.at[pl.ds(r0 + b * DR, DR)])
        plsc.subcore_barrier()

        eb = wid * EC

        def step(i, carry):
            off = eb + i * K
            pltpu.sync_copy(src_hbm.at[pl.ds(off, K)], isv)
            pltpu.sync_copy(dst_hbm.at[pl.ds(off, K)], idv)
            pltpu.async_copy(m_hbm.at[isv], rows, sem).wait()
            pltpu.sync_copy(rows, acc.at[idv], add=True)
            return carry
        lax.fori_loop(0, STEPS, step, 0)

        plsc.subcore_barrier()
        for b in range(RPT // DR):
            rr = r0 + b * DR
            pltpu.sync_copy(acc.at[pl.ds(rr, DR)], dbuf)
            pltpu.sync_copy(dbuf, out_hbm.at[c, pl.ds(rr, DR)])

    return k


def _rsqrt_body(d0_ref, d1_ref, o_ref):
    o_ref[...] = 1.0 / jnp.sqrt(d0_ref[...] + d1_ref[...] + 1.0)


def _dinv_tc(deg0, deg1):
    return pl.pallas_call(
        _rsqrt_body,
        out_shape=jax.ShapeDtypeStruct(deg0.shape, jnp.float32),
    )(deg0, deg1)


def _mm1_body(x_ref, dv_ref, w_ref, we_ref, be_ref, m_ref, re_ref):
    xb = x_ref[...]
    m_ref[...] = jnp.dot(xb, w_ref[...],
                         preferred_element_type=jnp.float32) * dv_ref[...]
    re_ref[...] = jnp.maximum(
        jnp.dot(xb, we_ref[...], preferred_element_type=jnp.float32)
        + be_ref[...], 0.0)


def _mm1_tc(x, dinv_b, W, We, be_r):
    N, F = x.shape
    R = 1000
    grid = (N // R,)
    blk = pl.BlockSpec((R, F), lambda i: (i, 0))
    wblk = pl.BlockSpec((F, F), lambda i: (0, 0))
    return pl.pallas_call(
        _mm1_body,
        grid=grid,
        in_specs=[blk, blk, wblk, wblk, pl.BlockSpec((1, F), lambda i: (0, 0))],
        out_specs=[blk, blk],
        out_shape=[jax.ShapeDtypeStruct((N, F), jnp.float32)] * 2,
    )(x, dinv_b, W, We, be_r)


def _comb_body(a0_ref, a1_ref, m_ref, re_ref, dv_ref, b_ref, w2_ref,
               h_ref, m2_ref):
    dv = dv_ref[...]
    s = (a0_ref[0] + a1_ref[0] + m_ref[...]) * dv + b_ref[...]
    h = jnp.maximum(s, 0.0) + re_ref[...]
    h_ref[...] = h
    m2_ref[...] = jnp.dot(h, w2_ref[...],
                          preferred_element_type=jnp.float32) * dv


def _comb_tc(acc, m, re, dinv_b, b_r, W2):
    N, F = m.shape
    R = 1000
    blk = pl.BlockSpec((R, F), lambda i: (i, 0))
    ablk0 = pl.BlockSpec((1, R, F), lambda i: (0, i, 0))
    ablk1 = pl.BlockSpec((1, R, F), lambda i: (1, i, 0))
    return pl.pallas_call(
        _comb_body,
        grid=(N // R,),
        in_specs=[ablk0, ablk1, blk, blk, blk,
                  pl.BlockSpec((1, F), lambda i: (0, 0)),
                  pl.BlockSpec((F, F), lambda i: (0, 0))],
        out_specs=[blk, blk],
        out_shape=[jax.ShapeDtypeStruct((N, F), jnp.float32)] * 2,
    )(acc, acc, m, re, dinv_b, b_r, W2)


def _final_body(a0_ref, a1_ref, m2_ref, h_ref, b2_ref,
                a0s_ref, a1s_ref, m2s_ref, hs_ref, b2s_ref,
                dv_ref, batch_ref, wf_ref, bf_ref, o_ref,
                seg_ref, cnt_ref):
    i = pl.program_id(0)
    n = pl.num_programs(0)
    G = seg_ref.shape[0]
    R = h_ref.shape[0]

    @pl.when(i == 0)
    def _():
        seg_ref[...] = jnp.zeros_like(seg_ref)
        cnt_ref[...] = jnp.zeros_like(cnt_ref)

    dv = dv_ref[...]
    h2 = jnp.maximum((a0_ref[0] + a1_ref[0] + m2_ref[...]) * dv
                     + b2_ref[...], 0.0) + h_ref[...]
    h2s = jnp.maximum((a0s_ref[0] + a1s_ref[0] + m2s_ref[...]) * dv
                      + b2s_ref[...], 0.0) + hs_ref[...]
    ht = h2 + h2s

    b = batch_ref[0]
    gid = lax.broadcasted_iota(jnp.int32, (G, R), 0)
    oh = (gid == jnp.broadcast_to(b, (G, R))).astype(jnp.float32)
    seg_ref[...] += jnp.dot(oh, ht, preferred_element_type=jnp.float32,
                         precision=lax.Precision.HIGHEST)
    cnt_ref[...] += jnp.broadcast_to(
        jnp.sum(oh, axis=1, keepdims=True), cnt_ref.shape)

    @pl.when(i == n - 1)
    def _():
        pooled = seg_ref[...] / jnp.maximum(cnt_ref[...], 1.0)
        pooled = pooled.astype(jnp.bfloat16).astype(jnp.float32)
        wf = wf_ref[...].astype(jnp.bfloat16).astype(jnp.float32)
        pred = jnp.sum(pooled * wf, axis=1, keepdims=True)
        o_ref[...] = pred + bf_ref[0, 0]


def _final_tc(acc2, m2, h, b2_r, acc2s, m2s, hs, b2s_r,
              dinv_b, batch3, wf_r, bf_b, G):
    N, F = h.shape
    R = 1000
    blk = pl.BlockSpec((R, F), lambda i: (i, 0))
    cblk = pl.BlockSpec((1, F), lambda i: (0, 0))
    ablk0 = pl.BlockSpec((1, R, F), lambda i: (0, i, 0))
    ablk1 = pl.BlockSpec((1, R, F), lambda i: (1, i, 0))
    return pl.pallas_call(
        _final_body,
        grid=(N // R,),
        in_specs=[ablk0, ablk1, blk, blk, cblk,
                  ablk0, ablk1, blk, blk, cblk,
                  blk,
                  pl.BlockSpec((1, 1, R), lambda i: (i, 0, 0)),
                  cblk, cblk],
        out_specs=pl.BlockSpec((G, 1), lambda i: (0, 0)),
        out_shape=jax.ShapeDtypeStruct((G, 1), jnp.float32),
        scratch_shapes=[pltpu.VMEM((G, F), jnp.float32),
                        pltpu.VMEM((G, F), jnp.float32)],
    )(acc2, acc2, m2, h, b2_r, acc2s, acc2s, m2s, hs, b2s_r, dinv_b, batch3,
      wf_r, bf_b)


def kernel(x, x_SC, edge_index, edge_weight, batch,
           W1, b1, W2, b2, We, be,
           W1s, b1s, W2s, b2s, Wes, bes,
           Wf, bf):
    N, F = x.shape
    E = edge_index.shape[1]
    G = 64
    NP = 10240

    src = edge_index[0].astype(jnp.int32)
    dst = edge_index[1].astype(jnp.int32)

    deg3 = _make_deg(E, NP)(dst)
    dinv80 = _dinv_tc(deg3[0, 0].reshape(NP // F, F),
                      deg3[1, 0].reshape(NP // F, F))
    dinv_b = jnp.broadcast_to(
        dinv80.reshape(NP)[:N][:, None], (N, F))

    spmm = _make_spmm(E, NP, F)

    m1, re1 = _mm1_tc(x, dinv_b, W1, We, be.reshape(1, F))
    m1s, re1s = _mm1_tc(x_SC, dinv_b, W1s, Wes, bes.reshape(1, F))

    acc1 = spmm(m1, src, dst)
    acc1s = spmm(m1s, src, dst)

    h1, m2 = _comb_tc(acc1, m1, re1, dinv_b, b1.reshape(1, F), W2)
    h1s, m2s = _comb_tc(acc1s, m1s, re1s, dinv_b, b1s.reshape(1, F), W2s)

    acc2 = spmm(m2, src, dst)
    acc2s = spmm(m2s, src, dst)

    batch3 = batch.astype(jnp.int32).reshape(N // 1000, 1, 1000)
    out = _final_tc(acc2, m2, h1, b2.reshape(1, F),
                    acc2s, m2s, h1s, b2s.reshape(1, F),
                    dinv_b, batch3, Wf.reshape(1, F),
                    jnp.broadcast_to(bf.reshape(1, 1), (1, F)), G)
    return out


# speedup vs baseline: 2.0159x; 1.1818x over previous
import functools

import jax
import jax.numpy as jnp
from jax import lax
from jax.experimental import pallas as pl
from jax.experimental.pallas import tpu as pltpu
from jax.experimental.pallas import tpu_sc as plsc

_NC = 2
_NS = 16
_NW = _NC * _NS


def _make_deg(E, NP):
    EC = E // _NW
    K = 80
    STEPS = EC // K
    NPS = NP // _NS
    mesh = plsc.VectorSubcoreMesh(core_axis_name="c", subcore_axis_name="s")

    @functools.partial(
        pl.kernel,
        out_type=jax.ShapeDtypeStruct((_NC, 1, NP), jnp.float32),
        mesh=mesh,
        scratch_types=[
            pltpu.VMEM((K,), jnp.int32),
            pltpu.VMEM((K,), jnp.float32),
            pltpu.VMEM((NPS,), jnp.float32),
            pltpu.VMEM_SHARED((NP,), jnp.float32),
        ],
    )
    def k(dst_hbm, deg_out, idv, onesv, dbuf, acc):
        c = lax.axis_index("c")
        s = lax.axis_index("s")
        wid = s * _NC + c

        for j in range(K // 16):
            onesv[pl.ds(j * 16, 16)] = jnp.ones((16,), jnp.float32)

        def zero(i, carry):
            dbuf[pl.ds(i * 16, 16)] = jnp.zeros((16,), jnp.float32)
            return carry
        lax.fori_loop(0, NPS // 16, zero, 0)
        pltpu.sync_copy(dbuf, acc.at[pl.ds(s * NPS, NPS)])
        plsc.subcore_barrier()

        def body(i, carry):
            off = wid * EC + i * K
            pltpu.sync_copy(dst_hbm.at[pl.ds(off, K)], idv)
            pltpu.sync_copy(onesv, acc.at[idv], add=True)
            return carry
        lax.fori_loop(0, STEPS, body, 0)

        plsc.subcore_barrier()
        pltpu.sync_copy(acc.at[pl.ds(s * NPS, NPS)], dbuf)
        pltpu.sync_copy(dbuf, deg_out.at[c, 0, pl.ds(s * NPS, NPS)])

    return k


def _make_spmm(E, NV2, F):
    EC = E // _NW
    K = 80
    STEPS = EC // K
    RPT = NV2 // _NS
    DR = 128
    mesh = plsc.VectorSubcoreMesh(core_axis_name="c", subcore_axis_name="s")

    @functools.partial(
        pl.kernel,
        out_type=jax.ShapeDtypeStruct((_NC, NV2, F), jnp.float32),
        mesh=mesh,
        scratch_types=[
            pltpu.VMEM((K,), jnp.int32),
            pltpu.VMEM((K,), jnp.int32),
            pltpu.VMEM((K, F), jnp.float32),
            pltpu.VMEM((DR, F), jnp.float32),
            pltpu.VMEM_SHARED((NV2, F), jnp.float32),
            pltpu.SemaphoreType.DMA,
        ],
    )
    def k(m_hbm, src_hbm, dst_hbm, out_hbm, isv, idv, rows, dbuf, acc,
          sem):
        c = lax.axis_index("c")
        s = lax.axis_index("s")
        wid = s * _NC + c

        def zero(i, carry):
            r = i // (F // 16)
            j = i - r * (F // 16)
            dbuf[r, pl.ds(j * 16, 16)] = jnp.zeros((16,), jnp.float32)
            return carry
        lax.fori_loop(0, DR * (F // 16), zero, 0)

        r0 = s * RPT
        for b in range(RPT // DR):
            pltpu.sync_copy(dbuf, acc.at[pl.ds(r0 + b * DR, DR)])
        plsc.subcore_barrier()

        eb = wid * EC

        def step(i, carry):
            off = eb + i * K
            pltpu.sync_copy(src_hbm.at[pl.ds(off, K)], isv)
            d = pltpu.async_copy(m_hbm.at[isv], rows, sem)
            pltpu.sync_copy(dst_hbm.at[pl.ds(off, K)], idv)
            d.wait()
            pltpu.sync_copy(rows, acc.at[idv], add=True)
            return carry
        lax.fori_loop(0, STEPS, step, 0)

        plsc.subcore_barrier()
        for b in range(RPT // DR):
            rr = r0 + b * DR
            pltpu.sync_copy(acc.at[pl.ds(rr, DR)], dbuf)
            pltpu.sync_copy(dbuf, out_hbm.at[c, pl.ds(rr, DR)])

    return k


def _rsqrt_body(d0_ref, d1_ref, o_ref):
    o_ref[...] = 1.0 / jnp.sqrt(d0_ref[...] + d1_ref[...] + 1.0)


def _dinv_tc(deg0, deg1):
    return pl.pallas_call(
        _rsqrt_body,
        out_shape=jax.ShapeDtypeStruct(deg0.shape, jnp.float32),
    )(deg0, deg1)


def _mm1_body(x_ref, dv_ref, w_ref, we_ref, be_ref, m_ref, re_ref):
    xb = x_ref[...]
    m_ref[...] = jnp.dot(xb, w_ref[...],
                         preferred_element_type=jnp.float32) * dv_ref[...]
    re_ref[...] = jnp.maximum(
        jnp.dot(xb, we_ref[...], preferred_element_type=jnp.float32)
        + be_ref[...], 0.0)


def _mm1_tc(x, dinv_b, W, We, be_r):
    N, F = x.shape
    R = 1000
    grid = (N // R,)
    blk = pl.BlockSpec((R, F), lambda i: (i, 0))
    wblk = pl.BlockSpec((F, F), lambda i: (0, 0))
    return pl.pallas_call(
        _mm1_body,
        grid=grid,
        in_specs=[blk, blk, wblk, wblk, pl.BlockSpec((1, F), lambda i: (0, 0))],
        out_specs=[blk, blk],
        out_shape=[jax.ShapeDtypeStruct((N, F), jnp.float32)] * 2,
    )(x, dinv_b, W, We, be_r)


def _comb_body(a0_ref, a1_ref, m_ref, re_ref, dv_ref, b_ref, w2_ref,
               h_ref, m2_ref):
    dv = dv_ref[...]
    s = (a0_ref[0] + a1_ref[0] + m_ref[...]) * dv + b_ref[...]
    h = jnp.maximum(s, 0.0) + re_ref[...]
    h_ref[...] = h
    m2_ref[...] = jnp.dot(h, w2_ref[...],
                          preferred_element_type=jnp.float32) * dv


def _comb_tc(acc, m, re, dinv_b, b_r, W2):
    N, F = m.shape
    R = 1000
    blk = pl.BlockSpec((R, F), lambda i: (i, 0))
    ablk0 = pl.BlockSpec((1, R, F), lambda i: (0, i, 0))
    ablk1 = pl.BlockSpec((1, R, F), lambda i: (1, i, 0))
    return pl.pallas_call(
        _comb_body,
        grid=(N // R,),
        in_specs=[ablk0, ablk1, blk, blk, blk,
                  pl.BlockSpec((1, F), lambda i: (0, 0)),
                  pl.BlockSpec((F, F), lambda i: (0, 0))],
        out_specs=[blk, blk],
        out_shape=[jax.ShapeDtypeStruct((N, F), jnp.float32)] * 2,
    )(acc, acc, m, re, dinv_b, b_r, W2)


def _final_body(a0_ref, a1_ref, m2_ref, h_ref, b2_ref,
                a0s_ref, a1s_ref, m2s_ref, hs_ref, b2s_ref,
                dv_ref, batch_ref, wf_ref, bf_ref, o_ref,
                seg_ref, cnt_ref):
    i = pl.program_id(0)
    n = pl.num_programs(0)
    G = seg_ref.shape[0]
    R = h_ref.shape[0]

    @pl.when(i == 0)
    def _():
        seg_ref[...] = jnp.zeros_like(seg_ref)
        cnt_ref[...] = jnp.zeros_like(cnt_ref)

    dv = dv_ref[...]
    h2 = jnp.maximum((a0_ref[0] + a1_ref[0] + m2_ref[...]) * dv
                     + b2_ref[...], 0.0) + h_ref[...]
    h2s = jnp.maximum((a0s_ref[0] + a1s_ref[0] + m2s_ref[...]) * dv
                      + b2s_ref[...], 0.0) + hs_ref[...]
    ht = h2 + h2s

    b = batch_ref[0]
    gid = lax.broadcasted_iota(jnp.int32, (G, R), 0)
    oh = (gid == jnp.broadcast_to(b, (G, R))).astype(jnp.float32)
    seg_ref[...] += jnp.dot(oh, ht, preferred_element_type=jnp.float32,
                         precision=lax.Precision.HIGHEST)
    cnt_ref[...] += jnp.broadcast_to(
        jnp.sum(oh, axis=1, keepdims=True), cnt_ref.shape)

    @pl.when(i == n - 1)
    def _():
        pooled = seg_ref[...] / jnp.maximum(cnt_ref[...], 1.0)
        pooled = pooled.astype(jnp.bfloat16).astype(jnp.float32)
        wf = wf_ref[...].astype(jnp.bfloat16).astype(jnp.float32)
        pred = jnp.sum(pooled * wf, axis=1, keepdims=True)
        o_ref[...] = pred + bf_ref[0, 0]


def _final_tc(acc2, m2, h, b2_r, acc2s, m2s, hs, b2s_r,
              dinv_b, batch3, wf_r, bf_b, G):
    N, F = h.shape
    R = 1000
    blk = pl.BlockSpec((R, F), lambda i: (i, 0))
    cblk = pl.BlockSpec((1, F), lambda i: (0, 0))
    ablk0 = pl.BlockSpec((1, R, F), lambda i: (0, i, 0))
    ablk1 = pl.BlockSpec((1, R, F), lambda i: (1, i, 0))
    return pl.pallas_call(
        _final_body,
        grid=(N // R,),
        in_specs=[ablk0, ablk1, blk, blk, cblk,
                  ablk0, ablk1, blk, blk, cblk,
                  blk,
                  pl.BlockSpec((1, 1, R), lambda i: (i, 0, 0)),
                  cblk, cblk],
        out_specs=pl.BlockSpec((G, 1), lambda i: (0, 0)),
        out_shape=jax.ShapeDtypeStruct((G, 1), jnp.float32),
        scratch_shapes=[pltpu.VMEM((G, F), jnp.float32),
                        pltpu.VMEM((G, F), jnp.float32)],
    )(acc2, acc2, m2, h, b2_r, acc2s, acc2s, m2s, hs, b2s_r, dinv_b, batch3,
      wf_r, bf_b)


def kernel(x, x_SC, edge_index, edge_weight, batch,
           W1, b1, W2, b2, We, be,
           W1s, b1s, W2s, b2s, Wes, bes,
           Wf, bf):
    N, F = x.shape
    E = edge_index.shape[1]
    G = 64
    NP = 10240

    src = edge_index[0].astype(jnp.int32)
    dst = edge_index[1].astype(jnp.int32)

    deg3 = _make_deg(E, NP)(dst)
    dinv80 = _dinv_tc(deg3[0, 0].reshape(NP // F, F),
                      deg3[1, 0].reshape(NP // F, F))
    dinv_b = jnp.broadcast_to(
        dinv80.reshape(NP)[:N][:, None], (N, F))

    spmm = _make_spmm(E, NP, F)

    m1, re1 = _mm1_tc(x, dinv_b, W1, We, be.reshape(1, F))
    m1s, re1s = _mm1_tc(x_SC, dinv_b, W1s, Wes, bes.reshape(1, F))

    acc1 = spmm(m1, src, dst)
    acc1s = spmm(m1s, src, dst)

    h1, m2 = _comb_tc(acc1, m1, re1, dinv_b, b1.reshape(1, F), W2)
    h1s, m2s = _comb_tc(acc1s, m1s, re1s, dinv_b, b1s.reshape(1, F), W2s)

    acc2 = spmm(m2, src, dst)
    acc2s = spmm(m2s, src, dst)

    batch3 = batch.astype(jnp.int32).reshape(N // 1000, 1, 1000)
    out = _final_tc(acc2, m2, h1, b2.reshape(1, F),
                    acc2s, m2s, h1s, b2s.reshape(1, F),
                    dinv_b, batch3, Wf.reshape(1, F),
                    jnp.broadcast_to(bf.reshape(1, 1), (1, F)), G)
    return out


# speedup vs baseline: 2.4329x; 1.2068x over previous
import functools

import jax
import jax.numpy as jnp
from jax import lax
from jax.experimental import pallas as pl
from jax.experimental.pallas import tpu as pltpu
from jax.experimental.pallas import tpu_sc as plsc

_NC = 2
_NS = 16
_NW = _NC * _NS


def _make_deg(E, NP):
    EC = E // _NW
    K = 80
    STEPS = EC // K
    NPS = NP // _NS
    mesh = plsc.VectorSubcoreMesh(core_axis_name="c", subcore_axis_name="s")

    @functools.partial(
        pl.kernel,
        out_type=jax.ShapeDtypeStruct((_NC, 1, NP), jnp.float32),
        mesh=mesh,
        scratch_types=[
            pltpu.VMEM((K,), jnp.int32),
            pltpu.VMEM((K,), jnp.float32),
            pltpu.VMEM((NPS,), jnp.float32),
            pltpu.VMEM_SHARED((NP,), jnp.float32),
        ],
    )
    def k(dst_hbm, deg_out, idv, onesv, dbuf, acc):
        c = lax.axis_index("c")
        s = lax.axis_index("s")
        wid = s * _NC + c

        for j in range(K // 16):
            onesv[pl.ds(j * 16, 16)] = jnp.ones((16,), jnp.float32)

        def zero(i, carry):
            dbuf[pl.ds(i * 16, 16)] = jnp.zeros((16,), jnp.float32)
            return carry
        lax.fori_loop(0, NPS // 16, zero, 0)
        pltpu.sync_copy(dbuf, acc.at[pl.ds(s * NPS, NPS)])
        plsc.subcore_barrier()

        def body(i, carry):
            off = wid * EC + i * K
            pltpu.sync_copy(dst_hbm.at[pl.ds(off, K)], idv)
            pltpu.sync_copy(onesv, acc.at[idv], add=True)
            return carry
        lax.fori_loop(0, STEPS, body, 0)

        plsc.subcore_barrier()
        pltpu.sync_copy(acc.at[pl.ds(s * NPS, NPS)], dbuf)
        pltpu.sync_copy(dbuf, deg_out.at[c, 0, pl.ds(s * NPS, NPS)])

    return k


def _make_spmm(E, NV2, F):
    EC = E // _NW
    K = 80
    STEPS = EC // K
    RPT = NV2 // _NS
    DR = 128
    mesh = plsc.VectorSubcoreMesh(core_axis_name="c", subcore_axis_name="s")

    @functools.partial(
        pl.kernel,
        out_type=jax.ShapeDtypeStruct((_NC, NV2, F), jnp.float32),
        mesh=mesh,
        scratch_types=[
            pltpu.VMEM((K,), jnp.int32),
            pltpu.VMEM((K,), jnp.int32),
            pltpu.VMEM((K,), jnp.int32),
            pltpu.VMEM((K,), jnp.int32),
            pltpu.VMEM((K, F), jnp.float32),
            pltpu.VMEM((DR, F), jnp.float32),
            pltpu.VMEM_SHARED((NV2, F), jnp.float32),
            pltpu.SemaphoreType.DMA,
        ],
    )
    def k(m_hbm, src_hbm, dst_hbm, out_hbm, isv0, isv1, idv0, idv1,
          rows, dbuf, acc, sem):
        c = lax.axis_index("c")
        s = lax.axis_index("s")
        wid = s * _NC + c

        def zero(i, carry):
            r = i // (F // 16)
            j = i - r * (F // 16)
            dbuf[r, pl.ds(j * 16, 16)] = jnp.zeros((16,), jnp.float32)
            return carry
        lax.fori_loop(0, DR * (F // 16), zero, 0)

        r0 = s * RPT
        for b in range(RPT // DR):
            pltpu.sync_copy(dbuf, acc.at[pl.ds(r0 + b * DR, DR)])
        plsc.subcore_barrier()

        eb = wid * EC

        pltpu.sync_copy(src_hbm.at[pl.ds(eb, K)], isv0)

        def nxt(i):
            return eb + lax.min(i, STEPS - 1) * K

        def step2(g, carry):
            i = 2 * g
            d = pltpu.async_copy(m_hbm.at[isv0], rows, sem)
            pltpu.sync_copy(dst_hbm.at[pl.ds(eb + i * K, K)], idv0)
            pltpu.sync_copy(src_hbm.at[pl.ds(nxt(i + 1), K)], isv1)
            d.wait()
            pltpu.sync_copy(rows, acc.at[idv0], add=True)
            d = pltpu.async_copy(m_hbm.at[isv1], rows, sem)
            pltpu.sync_copy(dst_hbm.at[pl.ds(eb + (i + 1) * K, K)], idv1)
            pltpu.sync_copy(src_hbm.at[pl.ds(nxt(i + 2), K)], isv0)
            d.wait()
            pltpu.sync_copy(rows, acc.at[idv1], add=True)
            return carry
        lax.fori_loop(0, STEPS // 2, step2, 0)

        d = pltpu.async_copy(m_hbm.at[isv0], rows, sem)
        pltpu.sync_copy(dst_hbm.at[pl.ds(eb + (STEPS - 1) * K, K)], idv0)
        d.wait()
        pltpu.sync_copy(rows, acc.at[idv0], add=True)

        plsc.subcore_barrier()
        for b in range(RPT // DR):
            rr = r0 + b * DR
            pltpu.sync_copy(acc.at[pl.ds(rr, DR)], dbuf)
            pltpu.sync_copy(dbuf, out_hbm.at[c, pl.ds(rr, DR)])

    return k


def _rsqrt_body(d0_ref, d1_ref, o_ref):
    o_ref[...] = 1.0 / jnp.sqrt(d0_ref[...] + d1_ref[...] + 1.0)


def _dinv_tc(deg0, deg1):
    return pl.pallas_call(
        _rsqrt_body,
        out_shape=jax.ShapeDtypeStruct(deg0.shape, jnp.float32),
    )(deg0, deg1)


def _mm1_body(x_ref, dv_ref, w_ref, we_ref, be_ref, m_ref, re_ref):
    xb = x_ref[...]
    m_ref[...] = jnp.dot(xb, w_ref[...],
                         preferred_element_type=jnp.float32) * dv_ref[...]
    re_ref[...] = jnp.maximum(
        jnp.dot(xb, we_ref[...], preferred_element_type=jnp.float32)
        + be_ref[...], 0.0)


def _mm1_tc(x, dinv_b, W, We, be_r):
    N, F = x.shape
    R = 1000
    grid = (N // R,)
    blk = pl.BlockSpec((R, F), lambda i: (i, 0))
    wblk = pl.BlockSpec((F, F), lambda i: (0, 0))
    return pl.pallas_call(
        _mm1_body,
        grid=grid,
        in_specs=[blk, blk, wblk, wblk, pl.BlockSpec((1, F), lambda i: (0, 0))],
        out_specs=[blk, blk],
        out_shape=[jax.ShapeDtypeStruct((N, F), jnp.float32)] * 2,
    )(x, dinv_b, W, We, be_r)


def _comb_body(a0_ref, a1_ref, m_ref, re_ref, dv_ref, b_ref, w2_ref,
               h_ref, m2_ref):
    dv = dv_ref[...]
    s = (a0_ref[0] + a1_ref[0] + m_ref[...]) * dv + b_ref[...]
    h = jnp.maximum(s, 0.0) + re_ref[...]
    h_ref[...] = h
    m2_ref[...] = jnp.dot(h, w2_ref[...],
                          preferred_element_type=jnp.float32) * dv


def _comb_tc(acc, m, re, dinv_b, b_r, W2):
    N, F = m.shape
    R = 1000
    blk = pl.BlockSpec((R, F), lambda i: (i, 0))
    ablk0 = pl.BlockSpec((1, R, F), lambda i: (0, i, 0))
    ablk1 = pl.BlockSpec((1, R, F), lambda i: (1, i, 0))
    return pl.pallas_call(
        _comb_body,
        grid=(N // R,),
        in_specs=[ablk0, ablk1, blk, blk, blk,
                  pl.BlockSpec((1, F), lambda i: (0, 0)),
                  pl.BlockSpec((F, F), lambda i: (0, 0))],
        out_specs=[blk, blk],
        out_shape=[jax.ShapeDtypeStruct((N, F), jnp.float32)] * 2,
    )(acc, acc, m, re, dinv_b, b_r, W2)


def _final_body(a0_ref, a1_ref, m2_ref, h_ref, b2_ref,
                a0s_ref, a1s_ref, m2s_ref, hs_ref, b2s_ref,
                dv_ref, batch_ref, wf_ref, bf_ref, o_ref,
                seg_ref, cnt_ref):
    i = pl.program_id(0)
    n = pl.num_programs(0)
    G = seg_ref.shape[0]
    R = h_ref.shape[0]

    @pl.when(i == 0)
    def _():
        seg_ref[...] = jnp.zeros_like(seg_ref)
        cnt_ref[...] = jnp.zeros_like(cnt_ref)

    dv = dv_ref[...]
    h2 = jnp.maximum((a0_ref[0] + a1_ref[0] + m2_ref[...]) * dv
                     + b2_ref[...], 0.0) + h_ref[...]
    h2s = jnp.maximum((a0s_ref[0] + a1s_ref[0] + m2s_ref[...]) * dv
                      + b2s_ref[...], 0.0) + hs_ref[...]
    ht = h2 + h2s

    b = batch_ref[0]
    gid = lax.broadcasted_iota(jnp.int32, (G, R), 0)
    oh = (gid == jnp.broadcast_to(b, (G, R))).astype(jnp.float32)
    seg_ref[...] += jnp.dot(oh, ht, preferred_element_type=jnp.float32,
                         precision=lax.Precision.HIGHEST)
    cnt_ref[...] += jnp.broadcast_to(
        jnp.sum(oh, axis=1, keepdims=True), cnt_ref.shape)

    @pl.when(i == n - 1)
    def _():
        pooled = seg_ref[...] / jnp.maximum(cnt_ref[...], 1.0)
        pooled = pooled.astype(jnp.bfloat16).astype(jnp.float32)
        wf = wf_ref[...].astype(jnp.bfloat16).astype(jnp.float32)
        pred = jnp.sum(pooled * wf, axis=1, keepdims=True)
        o_ref[...] = pred + bf_ref[0, 0]


def _final_tc(acc2, m2, h, b2_r, acc2s, m2s, hs, b2s_r,
              dinv_b, batch3, wf_r, bf_b, G):
    N, F = h.shape
    R = 1000
    blk = pl.BlockSpec((R, F), lambda i: (i, 0))
    cblk = pl.BlockSpec((1, F), lambda i: (0, 0))
    ablk0 = pl.BlockSpec((1, R, F), lambda i: (0, i, 0))
    ablk1 = pl.BlockSpec((1, R, F), lambda i: (1, i, 0))
    return pl.pallas_call(
        _final_body,
        grid=(N // R,),
        in_specs=[ablk0, ablk1, blk, blk, cblk,
                  ablk0, ablk1, blk, blk, cblk,
                  blk,
                  pl.BlockSpec((1, 1, R), lambda i: (i, 0, 0)),
                  cblk, cblk],
        out_specs=pl.BlockSpec((G, 1), lambda i: (0, 0)),
        out_shape=jax.ShapeDtypeStruct((G, 1), jnp.float32),
        scratch_shapes=[pltpu.VMEM((G, F), jnp.float32),
                        pltpu.VMEM((G, F), jnp.float32)],
    )(acc2, acc2, m2, h, b2_r, acc2s, acc2s, m2s, hs, b2s_r, dinv_b, batch3,
      wf_r, bf_b)


def kernel(x, x_SC, edge_index, edge_weight, batch,
           W1, b1, W2, b2, We, be,
           W1s, b1s, W2s, b2s, Wes, bes,
           Wf, bf):
    N, F = x.shape
    E = edge_index.shape[1]
    G = 64
    NP = 10240

    src = edge_index[0].astype(jnp.int32)
    dst = edge_index[1].astype(jnp.int32)

    deg3 = _make_deg(E, NP)(dst)
    dinv80 = _dinv_tc(deg3[0, 0].reshape(NP // F, F),
                      deg3[1, 0].reshape(NP // F, F))
    dinv_b = jnp.broadcast_to(
        dinv80.reshape(NP)[:N][:, None], (N, F))

    spmm = _make_spmm(E, NP, F)

    m1, re1 = _mm1_tc(x, dinv_b, W1, We, be.reshape(1, F))
    m1s, re1s = _mm1_tc(x_SC, dinv_b, W1s, Wes, bes.reshape(1, F))

    acc1 = spmm(m1, src, dst)
    acc1s = spmm(m1s, src, dst)

    h1, m2 = _comb_tc(acc1, m1, re1, dinv_b, b1.reshape(1, F), W2)
    h1s, m2s = _comb_tc(acc1s, m1s, re1s, dinv_b, b1s.reshape(1, F), W2s)

    acc2 = spmm(m2, src, dst)
    acc2s = spmm(m2s, src, dst)

    batch3 = batch.astype(jnp.int32).reshape(N // 1000, 1, 1000)
    out = _final_tc(acc2, m2, h1, b2.reshape(1, F),
                    acc2s, m2s, h1s, b2s.reshape(1, F),
                    dinv_b, batch3, Wf.reshape(1, F),
                    jnp.broadcast_to(bf.reshape(1, 1), (1, F)), G)
    return out
